# baseline (speedup 1.0000x reference)
import jax
import jax.numpy as jnp
from jax.experimental import pallas as pl
from jax.experimental.pallas import tpu as pltpu

N = 256
D = 256
OUT_DIM = 256
NUM_HEADS = 8
HEAD_DIM = 32
HH = NUM_HEADS * HEAD_DIM
NEG_SLOPE = 0.2
LARGE_NEGATIVE_BIAS = -1e9


def _gat_dense_kernel(x_ref, e_ref, f_row_ref,
                      Wn_ref, We_ref, Wg_ref, Wm_hbm, Wo_hbm, b_ref,
                      attn_ref, vm_row_ref, out_ref,
                      Wm_s, Wo_s, sem_m, sem_o):
    f32 = jnp.float32

    cp_m = pltpu.make_async_copy(Wm_hbm, Wm_s, sem_m)
    cp_m.start()
    cp_o = pltpu.make_async_copy(Wo_hbm, Wo_s, sem_o)
    cp_o.start()

    Hn = jnp.dot(x_ref[...], Wn_ref[...], preferred_element_type=f32)
    He = jnp.dot(e_ref[...], We_ref[...], preferred_element_type=f32)
    Hg = jnp.dot(f_row_ref[...], Wg_ref[...], preferred_element_type=f32)
    HnT = Hn.T
    HeT = He.T
    HgT = Hg.T

    vm_row = vm_row_ref[...].astype(f32)
    vm_col = vm_row.T
    mask_bias = (vm_col * vm_row - 1.0) * (-LARGE_NEGATIVE_BIAS)

    attn = attn_ref[...]
    k_of = jax.lax.broadcasted_iota(jnp.int32, (HH, NUM_HEADS), 0) // HEAD_DIM
    h_of = jax.lax.broadcasted_iota(jnp.int32, (HH, NUM_HEADS), 1)
    mask = (k_of == h_of).astype(f32)
    kT = jax.lax.broadcasted_iota(jnp.int32, (NUM_HEADS, HH), 1) // HEAD_DIM
    hT = jax.lax.broadcasted_iota(jnp.int32, (NUM_HEADS, HH), 0)
    maskT = (kT == hT).astype(f32)
    kk = jax.lax.broadcasted_iota(jnp.int32, (HEAD_DIM, HH), 1)
    dd = jax.lax.broadcasted_iota(jnp.int32, (HEAD_DIM, HH), 0)
    sel = ((kk - (kk // HEAD_DIM) * HEAD_DIM) == dd).astype(f32)

    def _row_flatten(part):
        spread = jnp.dot(part, sel, preferred_element_type=f32)
        return jnp.sum(spread * maskT, axis=0, keepdims=True)

    asrc_row = _row_flatten(attn[:, 0:HEAD_DIM])
    atgt_row = _row_flatten(attn[:, HEAD_DIM:2 * HEAD_DIM])
    aedg_row = _row_flatten(attn[:, 2 * HEAD_DIM:3 * HEAD_DIM])
    aglb_row = _row_flatten(attn[:, 3 * HEAD_DIM:])

    t_all = jnp.dot(Hn * atgt_row, mask, preferred_element_type=f32)
    c_all = jnp.dot(Hg * aglb_row, mask, preferred_element_type=f32)
    tp = t_all + c_all
    uT_all = (jnp.dot(maskT * asrc_row, HnT, preferred_element_type=f32)
              + jnp.dot(maskT * aedg_row, HeT, preferred_element_type=f32))

    cp_m.wait()
    Wm = Wm_s[...]
    Wm_n = jnp.concatenate(
        [Wm[h * 3 * HEAD_DIM:h * 3 * HEAD_DIM + HEAD_DIM] for h in range(NUM_HEADS)])
    Wm_e = jnp.concatenate(
        [Wm[h * 3 * HEAD_DIM + HEAD_DIM:h * 3 * HEAD_DIM + 2 * HEAD_DIM] for h in range(NUM_HEADS)])
    Wm_g = jnp.concatenate(
        [Wm[h * 3 * HEAD_DIM + 2 * HEAD_DIM:(h + 1) * 3 * HEAD_DIM] for h in range(NUM_HEADS)])
    M = (jnp.dot(Hn, Wm_n, preferred_element_type=f32)
         + jnp.dot(He, Wm_e, preferred_element_type=f32)
         + jnp.dot(Hg, Wm_g, preferred_element_type=f32))

    aggs = []
    for h in range(NUM_HEADS):
        sl = slice(h * HEAD_DIM, (h + 1) * HEAD_DIM)
        L = tp[:, h:h + 1] + uT_all[h:h + 1, :]
        L = jnp.maximum(L, NEG_SLOPE * L) + mask_bias
        m = jnp.max(L, axis=1, keepdims=True)
        p = jnp.exp(L - m)
        denom = jnp.sum(p, axis=1, keepdims=True)
        alpha = p / (denom + 1e-9)
        aggs.append(jnp.dot(alpha, M[:, sl], preferred_element_type=f32))
    agg = jnp.concatenate(aggs, axis=1)

    cp_o.wait()
    out = jnp.dot(agg, Wo_s[...], preferred_element_type=f32) + b_ref[...]
    out_ref[...] = out * vm_col


def kernel(x, e, f, valid_mask, W_node, W_edge, W_glob, W_msg, W_out, b_out,
           attn_vec):
    Nn = x.shape[0]
    dt = x.dtype
    return pl.pallas_call(
        _gat_dense_kernel,
        out_shape=jax.ShapeDtypeStruct((Nn, OUT_DIM), dt),
        in_specs=[
            pl.BlockSpec(memory_space=pltpu.MemorySpace.HBM) if i in (6, 7)
            else pl.BlockSpec((s1, s2), lambda: (0, 0))
            for i, (s1, s2) in enumerate([
                (Nn, D), (Nn, D), (1, D), (D, HH), (D, HH), (D, HH),
                (NUM_HEADS * 3 * HEAD_DIM, HH), (HH, OUT_DIM),
                (1, OUT_DIM), (NUM_HEADS, 4 * HEAD_DIM), (1, Nn)])
        ],
        scratch_shapes=[
            pltpu.VMEM((NUM_HEADS * 3 * HEAD_DIM, HH), jnp.float32),
            pltpu.VMEM((HH, OUT_DIM), jnp.float32),
            pltpu.SemaphoreType.DMA,
            pltpu.SemaphoreType.DMA,
        ],
    )(x, e, f.reshape(1, D),
      W_node, W_edge, W_glob, W_msg, W_out, b_out.reshape(1, OUT_DIM),
      attn_vec, valid_mask.reshape(1, Nn))


# speedup vs baseline: 1.1156x; 1.1156x over previous
import jax
import jax.numpy as jnp
from jax.experimental import pallas as pl

N = 256
D = 256
OUT_DIM = 256
NUM_HEADS = 8
HEAD_DIM = 32
HH = NUM_HEADS * HEAD_DIM
NEG_SLOPE = 0.2
LARGE_NEGATIVE_BIAS = -1e9


def _gat_dense_kernel(x_ref, e_ref, f_row_ref,
                      Wn_ref, We_ref, Wg_ref, Wm_ref, Wo_ref, b_ref,
                      attn_ref, vm_row_ref, out_ref):
    f32 = jnp.float32

    Hn = jnp.dot(x_ref[...], Wn_ref[...], preferred_element_type=f32)
    He = jnp.dot(e_ref[...], We_ref[...], preferred_element_type=f32)
    Hg = jnp.dot(f_row_ref[...], Wg_ref[...], preferred_element_type=f32)
    HnT = Hn.T
    HeT = He.T
    HgT = Hg.T

    vm_row = vm_row_ref[...].astype(f32)
    vm_col = vm_row.T
    mask_bias = (vm_col * vm_row - 1.0) * (-LARGE_NEGATIVE_BIAS)

    Wm = Wm_ref[...]
    Wm_n = jnp.concatenate(
        [Wm[h * 3 * HEAD_DIM:h * 3 * HEAD_DIM + HEAD_DIM] for h in range(NUM_HEADS)])
    Wm_e = jnp.concatenate(
        [Wm[h * 3 * HEAD_DIM + HEAD_DIM:h * 3 * HEAD_DIM + 2 * HEAD_DIM] for h in range(NUM_HEADS)])
    Wm_g = jnp.concatenate(
        [Wm[h * 3 * HEAD_DIM + 2 * HEAD_DIM:(h + 1) * 3 * HEAD_DIM] for h in range(NUM_HEADS)])
    M = (jnp.dot(Hn, Wm_n, preferred_element_type=f32)
         + jnp.dot(He, Wm_e, preferred_element_type=f32)
         + jnp.dot(Hg, Wm_g, preferred_element_type=f32))

    attn = attn_ref[...]
    k_of = jax.lax.broadcasted_iota(jnp.int32, (HH, NUM_HEADS), 0) // HEAD_DIM
    h_of = jax.lax.broadcasted_iota(jnp.int32, (HH, NUM_HEADS), 1)
    mask = (k_of == h_of).astype(f32)
    kT = jax.lax.broadcasted_iota(jnp.int32, (NUM_HEADS, HH), 1) // HEAD_DIM
    hT = jax.lax.broadcasted_iota(jnp.int32, (NUM_HEADS, HH), 0)
    maskT = (kT == hT).astype(f32)
    kk = jax.lax.broadcasted_iota(jnp.int32, (HEAD_DIM, HH), 1)
    dd = jax.lax.broadcasted_iota(jnp.int32, (HEAD_DIM, HH), 0)
    sel = ((kk - (kk // HEAD_DIM) * HEAD_DIM) == dd).astype(f32)

    def _row_flatten(part):
        spread = jnp.dot(part, sel, preferred_element_type=f32)
        return jnp.sum(spread * maskT, axis=0, keepdims=True)

    asrc_row = _row_flatten(attn[:, 0:HEAD_DIM])
    atgt_row = _row_flatten(attn[:, HEAD_DIM:2 * HEAD_DIM])
    aedg_row = _row_flatten(attn[:, 2 * HEAD_DIM:3 * HEAD_DIM])
    aglb_row = _row_flatten(attn[:, 3 * HEAD_DIM:])

    t_all = jnp.dot(Hn * atgt_row, mask, preferred_element_type=f32)
    c_all = jnp.dot(Hg * aglb_row, mask, preferred_element_type=f32)
    tp = t_all + c_all
    uT_all = (jnp.dot(maskT * asrc_row, HnT, preferred_element_type=f32)
              + jnp.dot(maskT * aedg_row, HeT, preferred_element_type=f32))

    aggs = []
    for h in range(NUM_HEADS):
        sl = slice(h * HEAD_DIM, (h + 1) * HEAD_DIM)
        L = tp[:, h:h + 1] + uT_all[h:h + 1, :]
        L = jnp.maximum(L, NEG_SLOPE * L) + mask_bias
        m = jnp.max(L, axis=1, keepdims=True)
        p = jnp.exp(L - m)
        denom = jnp.sum(p, axis=1, keepdims=True)
        alpha = p / (denom + 1e-9)
        aggs.append(jnp.dot(alpha, M[:, sl], preferred_element_type=f32))
    agg = jnp.concatenate(aggs, axis=1)

    out = jnp.dot(agg, Wo_ref[...], preferred_element_type=f32) + b_ref[...]
    out_ref[...] = out * vm_col


def kernel(x, e, f, valid_mask, W_node, W_edge, W_glob, W_msg, W_out, b_out,
           attn_vec):
    Nn = x.shape[0]
    dt = x.dtype
    return pl.pallas_call(
        _gat_dense_kernel,
        out_shape=jax.ShapeDtypeStruct((Nn, OUT_DIM), dt),
    )(x, e, f.reshape(1, D),
      W_node, W_edge, W_glob, W_msg, W_out, b_out.reshape(1, OUT_DIM),
      attn_vec, valid_mask.reshape(1, Nn))
